# baseline (speedup 1.0000x reference)
import functools

import jax
import jax.numpy as jnp
from jax import lax
from jax.experimental import pallas as pl
from jax.experimental.pallas import tpu as pltpu
from jax.experimental.pallas import tpu_sc as plsc

B = 4096
H = 50
D = 64
HP = 56
DP = 128
NC = 2
NS = 16
NW = NC * NS
BATCH_PW = B // NW
CHUNK_B = 4
NROUND = BATCH_PW // CHUNK_B


@jax.jit
def _sc_gather(x, table):
    mesh = plsc.VectorSubcoreMesh(core_axis_name="c", subcore_axis_name="s")

    @functools.partial(
        pl.kernel,
        mesh=mesh,
        out_type=jax.ShapeDtypeStruct((B, HP, DP), jnp.float32),
        scratch_types=[
            pltpu.VMEM((BATCH_PW, HP), jnp.int32),
            pltpu.VMEM((2, CHUNK_B, HP, D), jnp.float32),
            pltpu.SemaphoreType.DMA,
        ],
        compiler_params=pltpu.CompilerParams(use_tc_tiling_on_sc=False),
    )
    def k(x_hbm, table_hbm, out_hbm, idx_v, rows_v, gsem):
        wid = lax.axis_index("s") * NC + lax.axis_index("c")
        bbase = wid * BATCH_PW
        pltpu.sync_copy(x_hbm.at[pl.ds(bbase, BATCH_PW), pl.ds(0, HP)], idx_v)

        def fire(r, slot):
            for b in range(CHUNK_B):
                pltpu.async_copy(
                    table_hbm.at[idx_v.at[r * CHUNK_B + b]],
                    rows_v.at[slot, b],
                    gsem,
                )

        def drain_and_store(r, slot):
            for b in range(CHUNK_B):
                pltpu.make_async_copy(
                    table_hbm.at[pl.ds(0, HP)], rows_v.at[slot, b], gsem
                ).wait()
            for b in range(CHUNK_B):
                pltpu.sync_copy(
                    rows_v.at[slot, b],
                    out_hbm.at[bbase + r * CHUNK_B + b, pl.ds(0, HP), pl.ds(0, D)],
                )

        fire(0, 0)

        def body(r, carry):
            fire(r + 1, lax.rem(r + 1, 2))
            drain_and_store(r, lax.rem(r, 2))
            return carry

        lax.fori_loop(0, NROUND - 1, body, 0)
        drain_and_store(NROUND - 1, (NROUND - 1) % 2)

    return k(x, table)


def kernel(x, table):
    xi = x.astype(jnp.int32)
    xp = jnp.concatenate(
        [xi, xi[:, : HP - H], jnp.zeros((B, 128 - HP - (HP - H)), jnp.int32) + xi[:, :1]],
        axis=1,
    )
    out = _sc_gather(xp, table)
    return out[:, :H, :D]


# speedup vs baseline: 1.0412x; 1.0412x over previous
import functools

import jax
import jax.numpy as jnp
from jax import lax
from jax.experimental import pallas as pl
from jax.experimental.pallas import tpu as pltpu
from jax.experimental.pallas import tpu_sc as plsc

B = 4096
H = 50
D = 64
HP = 56
DP = 128
N = B * H
NC = 2
NS = 16
NW = NC * NS
BATCH_PW = B // NW
ROWS_PW = BATCH_PW * H
CHUNK_B = 8
CHUNK = CHUNK_B * H
NROUND = BATCH_PW // CHUNK_B


@jax.jit
def _sc_gather(x1, table):
    mesh = plsc.VectorSubcoreMesh(core_axis_name="c", subcore_axis_name="s")

    @functools.partial(
        pl.kernel,
        mesh=mesh,
        out_type=jax.ShapeDtypeStruct((B, HP, DP), jnp.float32),
        scratch_types=[
            pltpu.VMEM((ROWS_PW,), jnp.int32),
            pltpu.VMEM((2, CHUNK, D), jnp.float32),
            pltpu.SemaphoreType.DMA,
        ],
        compiler_params=pltpu.CompilerParams(use_tc_tiling_on_sc=False),
    )
    def k(x_hbm, table_hbm, out_hbm, idx_v, rows_v, gsem):
        wid = lax.axis_index("s") * NC + lax.axis_index("c")
        bbase = wid * BATCH_PW
        pltpu.sync_copy(x_hbm.at[pl.ds(wid * ROWS_PW, ROWS_PW)], idx_v)

        def fire(r, slot):
            pltpu.async_copy(
                table_hbm.at[idx_v.at[pl.ds(r * CHUNK, CHUNK)]],
                rows_v.at[slot],
                gsem,
            )

        def drain_and_store(r, slot):
            pltpu.make_async_copy(
                table_hbm.at[pl.ds(0, CHUNK)], rows_v.at[slot], gsem
            ).wait()
            for b in range(CHUNK_B):
                pltpu.sync_copy(
                    rows_v.at[slot, pl.ds(b * H, H)],
                    out_hbm.at[bbase + r * CHUNK_B + b, pl.ds(0, H), pl.ds(0, D)],
                )

        fire(0, 0)

        def body(r, carry):
            fire(r + 1, lax.rem(r + 1, 2))
            drain_and_store(r, lax.rem(r, 2))
            return carry

        lax.fori_loop(0, NROUND - 1, body, 0)
        drain_and_store(NROUND - 1, (NROUND - 1) % 2)

    return k(x1, table)


def kernel(x, table):
    out = _sc_gather(x.astype(jnp.int32).reshape(N), table)
    return out[:, :H, :D]
